# baseline (speedup 1.0000x reference)
import functools

import jax
import jax.numpy as jnp
from jax import lax
from jax.experimental import pallas as pl
from jax.experimental.pallas import tpu as pltpu
from jax.experimental.pallas import tpu_sc as plsc

_NC = 2
_NS = 16
_NW = _NC * _NS
_LANES = 16


@functools.cache
def _gather_pool_kernel(n: int, v: int, d: int):
    dh = d // 2
    chunk = 128
    sc = 2 * chunk
    blk_sc = 4
    stage = blk_sc * sc
    per_w = n // _NW
    n_sc = per_w // sc
    n_blk = n_sc // blk_sc
    assert per_w * _NW == n and n_blk * stage == per_w
    assert n_sc % 2 == 0 and n_blk >= 3

    mesh = plsc.VectorSubcoreMesh(
        core_axis_name="c", subcore_axis_name="s",
        num_cores=_NC, num_subcores=_NS,
    )

    @functools.partial(
        pl.kernel,
        out_type=(
            jax.ShapeDtypeStruct((n, d), jnp.float32),
            jax.ShapeDtypeStruct((n * dh,), jnp.float32),
        ),
        mesh=mesh,
        scratch_types=[
            pltpu.VMEM((2, stage), jnp.int32),
            pltpu.VMEM((2, sc, d), jnp.float32),
            pltpu.VMEM((2, sc * dh), jnp.float32),
            pltpu.SemaphoreType.DMA,
        ] + [pltpu.SemaphoreType.DMA] * 6,
        compiler_params=pltpu.CompilerParams(needs_layout_passes=False),
    )
    def gather_k(idx_hbm, t_hbm, x_hbm, p_hbm, idxb, xb, pb, sem_i, *sems):
        sem_g, sem_wx, sem_wp = sems[:2], sems[2:4], sems[4:6]
        wid = lax.axis_index("s") * _NC + lax.axis_index("c")
        base = wid * per_w
        lane = lax.iota(jnp.int32, _LANES)

        def issue_gathers(tb, sl, grp):
            for ci in range(2):
                iv = idxb.at[tb, pl.ds((2 * sl + ci) * chunk, chunk)]
                pltpu.async_copy(t_hbm.at[iv], xb.at[grp, pl.ds(ci * chunk, chunk)],
                                 sem_g[grp])

        def wait_gathers(grp):
            for ci in range(2):
                iv = idxb.at[0, pl.ds(0, chunk)]
                pltpu.make_async_copy(
                    t_hbm.at[iv], xb.at[grp, pl.ds(ci * chunk, chunk)],
                    sem_g[grp]).wait()

        def wait_wx(grp):
            pltpu.make_async_copy(
                xb.at[grp], x_hbm.at[pl.ds(0, sc)], sem_wx[grp]).wait()

        def wait_wp(grp):
            pltpu.make_async_copy(
                pb.at[grp], p_hbm.at[pl.ds(0, sc * dh)], sem_wp[grp]).wait()

        def stage_idx(b, tb):
            pltpu.async_copy(
                idx_hbm.at[pl.ds(base + b * stage, stage)], idxb.at[tb], sem_i)

        def wait_idx():
            pltpu.make_async_copy(
                idx_hbm.at[pl.ds(0, stage)], idxb.at[0], sem_i).wait()

        def pool(grp):
            def pool_row(r):
                rvec = jnp.broadcast_to(r, (_LANES,))
                for c in range(dh // _LANES):
                    ev = 32 * c + 2 * lane
                    e = plsc.load_gather(xb.at[grp], [rvec, ev])
                    o = plsc.load_gather(xb.at[grp], [rvec, ev + 1])
                    pb[grp, pl.ds(r * dh + c * _LANES, _LANES)] = (
                        jnp.maximum(e, o))
            pl.loop(0, sc)(pool_row)

        pltpu.sync_copy(idx_hbm.at[pl.ds(base, stage)], idxb.at[0])
        issue_gathers(0, 0, 0)
        stage_idx(1, 1)

        def blk_body(b):
            tb = lax.rem(b, 2)
            for u in range(blk_sc):
                grp = u % 2
                wait_gathers(grp)
                if u == 0:
                    def pf_first(tb=tb):
                        wait_wx(1)
                        issue_gathers(tb, 1, 1)
                    pl.when(b > 0)(pf_first)
                    pl.when(b == 0)(lambda tb=tb: issue_gathers(tb, 1, 1))
                elif u < blk_sc - 1:
                    def pf_mid(tb=tb, u=u):
                        wait_wx((u + 1) % 2)
                        issue_gathers(tb, u + 1, (u + 1) % 2)
                    pf_mid()
                else:
                    def pf_last(tb=tb):
                        wait_wx((blk_sc) % 2)
                        wait_idx()
                        issue_gathers(1 - tb, 0, blk_sc % 2)
                    pl.when(b < n_blk - 1)(pf_last)
                if u < 2:
                    pl.when(b > 0)(lambda grp=grp: wait_wp(grp))
                else:
                    wait_wp(grp)
                pool(grp)
                off = base + (b * blk_sc + u) * sc
                pltpu.async_copy(xb.at[grp], x_hbm.at[pl.ds(off, sc)],
                                 sem_wx[grp])
                pltpu.async_copy(pb.at[grp], p_hbm.at[pl.ds(off * dh, sc * dh)],
                                 sem_wp[grp])
            pl.when(b < n_blk - 2)(lambda tb=tb: stage_idx(b + 2, tb))

        pl.loop(0, n_blk)(blk_body)
        for grp in range(2):
            wait_wx(grp)
            wait_wp(grp)

    return gather_k


def kernel(inputs, table):
    b, l = inputs.shape
    v, d = table.shape
    n = b * l
    x_flat, p_flat = _gather_pool_kernel(n, v, d)(inputs.reshape(-1), table)
    return x_flat.reshape(b, l, d), p_flat.reshape(b, l, d // 2)
